# baseline (speedup 1.0000x reference)
import functools

import jax
import jax.numpy as jnp
from jax import lax
from jax.experimental import pallas as pl
from jax.experimental.pallas import tpu as pltpu
from jax.experimental.pallas import tpu_sc as plsc

N_KEEP = 1000
N_PAD = 1024
ROW_TILE = 16
COL_CHUNK = 128
N_RT = N_PAD // ROW_TILE
N_CC = N_PAD // COL_CHUNK

OFF_W1 = 0
OFF_B1 = 224
OFF_W2 = 256
OFF_B2 = 768
OFF_W3 = 784
OFF_B3 = 800
OFF_L1 = 801
OFF_LB1 = 881
OFF_L2 = 897
OFF_LB2 = 913
N_TBL = 914


def _dense_kernel(do_mlp_ref,
                  boxes_r_ref, boxesT_ref, scores_r_ref, scoresT_ref,
                  classes_r_ref, classesT_ref, tbl_ref,
                  out_ref, s_plane_ref):
    i = pl.program_id(0)

    ones = jnp.ones((ROW_TILE, COL_CHUNK), jnp.float32)
    x1r = boxes_r_ref[:, 0:1] * ones
    y1r = boxes_r_ref[:, 1:2] * ones
    x2r = boxes_r_ref[:, 2:3] * ones
    y2r = boxes_r_ref[:, 3:4] * ones
    s_r = scores_r_ref[...] * ones
    c_r = classes_r_ref[...] * jnp.ones((ROW_TILE, COL_CHUNK), jnp.int32)
    area_r = (x2r - x1r) * (y2r - y1r)

    row_ids = i * ROW_TILE + jax.lax.broadcasted_iota(
        jnp.int32, (ROW_TILE, COL_CHUNK), 0)

    def wt(idx):
        return tbl_ref[idx]

    s_plane_ref[...] = jnp.zeros((ROW_TILE, COL_CHUNK), jnp.float32)
    D_plane = jnp.zeros((ROW_TILE, COL_CHUNK), jnp.float32)

    for chunk in range(N_CC):
        c0 = chunk * COL_CHUNK
        x1c = boxesT_ref[0:1, c0:c0 + COL_CHUNK] * ones
        y1c = boxesT_ref[1:2, c0:c0 + COL_CHUNK] * ones
        x2c = boxesT_ref[2:3, c0:c0 + COL_CHUNK] * ones
        y2c = boxesT_ref[3:4, c0:c0 + COL_CHUNK] * ones
        s_c = scoresT_ref[0:1, c0:c0 + COL_CHUNK] * ones
        area_c = (x2c - x1c) * (y2c - y1c)

        w = jnp.maximum(jnp.minimum(x2r, x2c) - jnp.maximum(x1r, x1c), 0.0)
        h = jnp.maximum(jnp.minimum(y2r, y2c) - jnp.maximum(y1r, y1c), 0.0)
        inter = w * h
        union = area_r + area_c - inter
        iou = inter / (union + 1e-06)

        col_ids = c0 + jax.lax.broadcasted_iota(
            jnp.int32, (ROW_TILE, COL_CHUNK), 1)
        iou = jnp.where(row_ids == col_ids, 0.0, iou)

        D_plane = D_plane + iou

        @pl.when(do_mlp_ref[i, chunk] != 0)
        def _():
            c_c = classesT_ref[0:1, c0:c0 + COL_CHUNK] * jnp.ones(
                (ROW_TILE, COL_CHUNK), jnp.int32)
            dx1 = jnp.abs(x1r - x1c)
            dy1 = jnp.abs(y1r - y1c)
            dx2 = jnp.abs(x2r - x2c)
            dy2 = jnp.abs(y2r - y2c)

            h2 = [wt(OFF_B2 + m) for m in range(16)]
            for k in range(32):
                t = wt(OFF_B1 + k)
                t = t + wt(OFF_W1 + 0 * 32 + k) * iou
                t = t + wt(OFF_W1 + 1 * 32 + k) * dx1
                t = t + wt(OFF_W1 + 2 * 32 + k) * dy1
                t = t + wt(OFF_W1 + 3 * 32 + k) * dx2
                t = t + wt(OFF_W1 + 4 * 32 + k) * dy2
                t = t + wt(OFF_W1 + 5 * 32 + k) * s_r
                t = t + wt(OFF_W1 + 6 * 32 + k) * s_c
                h1k = jnp.maximum(t, 0.0)
                for m in range(16):
                    h2[m] = h2[m] + wt(OFF_W2 + k * 16 + m) * h1k
            s_pre = wt(OFF_B3)
            for m in range(16):
                s_pre = s_pre + wt(OFF_W3 + m) * jnp.maximum(h2[m], 0.0)
            s_ij = jax.nn.sigmoid(s_pre)

            mask = jnp.logical_and(c_r == c_c, s_c > s_r)
            contrib = jnp.where(mask, s_ij * iou, 0.0)
            s_plane_ref[...] += contrib

    D = jnp.sum(D_plane, axis=1, keepdims=True) * (1.0 / N_KEEP)
    S = jnp.sum(s_plane_ref[...], axis=1, keepdims=True)

    lam_cols = (x1r, y1r, x2r, y2r, s_r)
    lam_pre = wt(OFF_LB2)
    for t in range(16):
        a = wt(OFF_LB1 + t)
        for c in range(5):
            a = a + wt(OFF_L1 + c * 16 + t) * lam_cols[c]
        lam_pre = lam_pre + wt(OFF_L2 + t) * jnp.maximum(a, 0.0)
    lam = jax.nn.sigmoid(lam_pre[:, 0:1])

    E = lam * S * D
    new_s = scores_r_ref[...] * jnp.exp(-E)
    out_ref[...] = jnp.where(row_ids[:, 0:1] < N_KEEP, new_s, -1.0)


NW = 16
EPW = N_PAD // NW
PAD_ROW0 = N_KEEP - (NW - 1) * EPW


def _sc_group_kernel(idx_hbm, scores_hbm, classes_hbm,
                     x1_hbm, y1_hbm, x2_hbm, y2_hbm,
                     x1_out, y1_out, x2_out, y2_out, scores_out, classes_out,
                     idx_v, sc_v, cls_v, bx_v, hist, lanepref, runhist,
                     mycnt, cc_v, cbase, dest_v, cc_sh, sem, sem2):
    wid = lax.axis_index("s")
    base = wid * EPW

    pltpu.sync_copy(idx_hbm.at[pl.ds(base, EPW)], idx_v)
    pltpu.sync_copy(scores_hbm.at[pl.ds(base, EPW)], sc_v)
    cps = [
        pltpu.async_copy(classes_hbm.at[idx_v], cls_v, sem),
        pltpu.async_copy(x1_hbm.at[idx_v], bx_v.at[0], sem),
        pltpu.async_copy(y1_hbm.at[idx_v], bx_v.at[1], sem),
        pltpu.async_copy(x2_hbm.at[idx_v], bx_v.at[2], sem),
        pltpu.async_copy(y2_hbm.at[idx_v], bx_v.at[3], sem),
    ]
    for c in cps:
        c.wait()

    for v in range(2048 // 16):
        hist[pl.ds(v * 16, 16)] = jnp.zeros((16,), jnp.int32)
        runhist[pl.ds(v * 16, 16)] = jnp.zeros((16,), jnp.int32)

    lane = lax.iota(jnp.int32, 16)
    ones16 = jnp.ones((16,), jnp.int32)
    zero16 = jnp.zeros((16,), jnp.float32)
    for v in range(EPW // 16):
        sl = pl.ds(v * 16, 16)
        pos = base + v * 16 + lane
        mpad = pos >= N_KEEP
        c16 = jnp.where(mpad, 127, cls_v[sl])
        cls_v[sl] = c16
        sc_v[sl] = jnp.where(mpad, -1.0, sc_v[sl])
        for d in range(4):
            bx_v[d, sl] = jnp.where(mpad, zero16, bx_v[d, sl])
        plsc.addupdate_scatter(hist, [lane * 128 + c16], ones16)

    for cb in range(128 // 16):
        acc = jnp.zeros((16,), jnp.int32)
        for l in range(16):
            lanepref[pl.ds(l * 128 + cb * 16, 16)] = acc
            acc = acc + hist[pl.ds(l * 128 + cb * 16, 16)]
        mycnt[pl.ds(cb * 16, 16)] = acc

    pltpu.sync_copy(mycnt, cc_sh.at[wid])
    plsc.subcore_barrier()
    pltpu.sync_copy(cc_sh, cc_v)

    carry = jnp.int32(0)
    for cb in range(128 // 16):
        tot_cb = jnp.zeros((16,), jnp.int32)
        pw_cb = jnp.zeros((16,), jnp.int32)
        for w in range(NW):
            row = cc_v[w, pl.ds(cb * 16, 16)]
            tot_cb = tot_cb + row
            pw_cb = pw_cb + row * jnp.int32(w < wid)
        incl = plsc.cumsum(tot_cb)
        excl = incl - tot_cb + carry
        cbase[pl.ds(cb * 16, 16)] = excl + pw_cb
        carry = carry + jnp.sum(tot_cb)

    for v in range(EPW // 16):
        c16 = cls_v[pl.ds(v * 16, 16)]
        lidx = lane * 128 + c16
        base16 = plsc.load_gather(cbase, [c16])
        lp16 = plsc.load_gather(lanepref, [lidx])
        rl16 = plsc.load_gather(runhist, [lidx])
        dest_v[pl.ds(v * 16, 16)] = base16 + lp16 + rl16
        plsc.addupdate_scatter(runhist, [lidx], ones16)

    cps = [
        pltpu.async_copy(bx_v.at[0], x1_out.at[dest_v], sem2),
        pltpu.async_copy(bx_v.at[1], y1_out.at[dest_v], sem2),
        pltpu.async_copy(bx_v.at[2], x2_out.at[dest_v], sem2),
        pltpu.async_copy(bx_v.at[3], y2_out.at[dest_v], sem2),
        pltpu.async_copy(sc_v, scores_out.at[dest_v], sem2),
        pltpu.async_copy(cls_v, classes_out.at[dest_v], sem2),
    ]
    for c in cps:
        c.wait()


def _sc_group(idx, scores_k, classes, boxes):
    idx_p = jnp.pad(idx, (0, N_PAD - N_KEEP))
    scores_p = jnp.pad(scores_k, (0, N_PAD - N_KEEP))
    f32 = jnp.float32
    outs = pl.kernel(
        _sc_group_kernel,
        out_type=(jax.ShapeDtypeStruct((N_PAD,), f32),
                  jax.ShapeDtypeStruct((N_PAD,), f32),
                  jax.ShapeDtypeStruct((N_PAD,), f32),
                  jax.ShapeDtypeStruct((N_PAD,), f32),
                  jax.ShapeDtypeStruct((N_PAD,), f32),
                  jax.ShapeDtypeStruct((N_PAD,), jnp.int32)),
        mesh=plsc.VectorSubcoreMesh(core_axis_name="c",
                                    subcore_axis_name="s",
                                    num_cores=1, num_subcores=NW),
        compiler_params=pltpu.CompilerParams(needs_layout_passes=False),
        scratch_types=[
            pltpu.VMEM((EPW,), jnp.int32),
            pltpu.VMEM((EPW,), jnp.float32),
            pltpu.VMEM((EPW,), jnp.int32),
            pltpu.VMEM((4, EPW), jnp.float32),
            pltpu.VMEM((2048,), jnp.int32),
            pltpu.VMEM((2048,), jnp.int32),
            pltpu.VMEM((2048,), jnp.int32),
            pltpu.VMEM((128,), jnp.int32),
            pltpu.VMEM((NW, 128), jnp.int32),
            pltpu.VMEM((128,), jnp.int32),
            pltpu.VMEM((EPW,), jnp.int32),
            pltpu.VMEM_SHARED((NW, 128), jnp.int32),
            pltpu.SemaphoreType.DMA,
            pltpu.SemaphoreType.DMA,
        ],
    )(idx_p, scores_p, classes,
      boxes[:, 0], boxes[:, 1], boxes[:, 2], boxes[:, 3])
    x1, y1, x2, y2, scores_s, classes_s = outs
    boxes_s = jnp.stack([x1, y1, x2, y2], axis=1)
    return boxes_s, scores_s, classes_s


@jax.jit
def _dense_stage(boxes_p, scores_p, classes_p,
                 W1, b1, W2, b2, W3, b3, L1, lb1, L2, lb2):
    cls_first = classes_p[::ROW_TILE]
    cls_last = classes_p[ROW_TILE - 1::ROW_TILE]
    ws = jnp.searchsorted(classes_p, cls_first, side="left")
    we = jnp.searchsorted(classes_p, cls_last, side="right")
    c_lo = jnp.arange(N_CC) * COL_CHUNK
    c_hi = c_lo + COL_CHUNK
    do_mlp = jnp.logical_and(c_hi[None, :] > ws[:, None],
                             c_lo[None, :] < we[:, None]).astype(jnp.int32)

    vals = jnp.concatenate([
        W1.reshape(-1), b1, W2.reshape(-1), b2, W3.reshape(-1), b3,
        L1.reshape(-1), lb1, L2.reshape(-1), lb2])
    tbl = jnp.broadcast_to(vals[:, None, None],
                           (N_TBL, ROW_TILE, COL_CHUNK))

    boxesT = boxes_p.T
    scores_r = scores_p[:, None]
    scoresT = scores_p[None, :]
    classes_r = classes_p[:, None]
    classesT = classes_p[None, :]

    grid = (N_RT,)
    row_spec2 = lambda w: pl.BlockSpec((ROW_TILE, w), lambda i: (i, 0))
    full = lambda a, b: pl.BlockSpec((a, b), lambda i: (0, 0))
    smem = pl.BlockSpec(memory_space=pltpu.SMEM)

    out = pl.pallas_call(
        _dense_kernel,
        grid=grid,
        in_specs=[
            smem,
            row_spec2(4),
            full(4, N_PAD),
            row_spec2(1),
            full(1, N_PAD),
            row_spec2(1),
            full(1, N_PAD),
            pl.BlockSpec((N_TBL, ROW_TILE, COL_CHUNK),
                         lambda i: (0, 0, 0)),
        ],
        out_specs=pl.BlockSpec((ROW_TILE, 1), lambda i: (i, 0)),
        out_shape=jax.ShapeDtypeStruct((N_PAD, 1), jnp.float32),
        scratch_shapes=[pltpu.VMEM((ROW_TILE, COL_CHUNK), jnp.float32)],
    )(do_mlp, boxes_p, boxesT, scores_r, scoresT, classes_r, classesT, tbl)
    return out[:N_KEEP, 0]


def kernel(boxes, scores, classes, W1, b1, W2, b2, W3, b3, L1, lb1, L2, lb2):
    scores_k, idx = jax.lax.top_k(scores, N_KEEP)

    boxes_s, scores_s, classes_s = _sc_group(idx, scores_k, classes, boxes)

    new_scores = _dense_stage(boxes_s, scores_s, classes_s,
                              W1, b1, W2, b2, W3, b3, L1, lb1, L2, lb2)
    _, idx2 = jax.lax.top_k(new_scores, 50)
    return (boxes_s[idx2], new_scores[idx2], classes_s[idx2])


# speedup vs baseline: 1.3124x; 1.3124x over previous
import functools

import jax
import jax.numpy as jnp
from jax import lax
from jax.experimental import pallas as pl
from jax.experimental.pallas import tpu as pltpu
from jax.experimental.pallas import tpu_sc as plsc

N_KEEP = 1000
N_PAD = 1024
ROW_TILE = 16
COL_CHUNK = 128
N_RT = N_PAD // ROW_TILE
N_CC = N_PAD // COL_CHUNK

OFF_W1 = 0
OFF_B1 = 224
OFF_W2 = 256
OFF_B2 = 768
OFF_W3 = 784
OFF_B3 = 800
OFF_L1 = 801
OFF_LB1 = 881
OFF_L2 = 897
OFF_LB2 = 913
N_TBL = 914


def _dense_kernel(do_mlp_ref,
                  boxes_r_ref, boxesT_ref, scores_r_ref, scoresT_ref,
                  classes_r_ref, classesT_ref, tbl_ref,
                  out_ref, s_plane_ref):
    i = pl.program_id(0)

    ones = jnp.ones((ROW_TILE, COL_CHUNK), jnp.float32)
    x1r = boxes_r_ref[:, 0:1] * ones
    y1r = boxes_r_ref[:, 1:2] * ones
    x2r = boxes_r_ref[:, 2:3] * ones
    y2r = boxes_r_ref[:, 3:4] * ones
    s_r = scores_r_ref[...] * ones
    c_r = classes_r_ref[...] * jnp.ones((ROW_TILE, COL_CHUNK), jnp.int32)
    area_r = (x2r - x1r) * (y2r - y1r)

    row_ids = i * ROW_TILE + jax.lax.broadcasted_iota(
        jnp.int32, (ROW_TILE, COL_CHUNK), 0)

    def wt(idx):
        return tbl_ref[idx]

    s_plane_ref[...] = jnp.zeros((ROW_TILE, COL_CHUNK), jnp.float32)
    D_plane = jnp.zeros((ROW_TILE, COL_CHUNK), jnp.float32)

    for chunk in range(N_CC):
        c0 = chunk * COL_CHUNK
        x1c = boxesT_ref[0:1, c0:c0 + COL_CHUNK] * ones
        y1c = boxesT_ref[1:2, c0:c0 + COL_CHUNK] * ones
        x2c = boxesT_ref[2:3, c0:c0 + COL_CHUNK] * ones
        y2c = boxesT_ref[3:4, c0:c0 + COL_CHUNK] * ones
        s_c = scoresT_ref[0:1, c0:c0 + COL_CHUNK] * ones
        area_c = (x2c - x1c) * (y2c - y1c)

        w = jnp.maximum(jnp.minimum(x2r, x2c) - jnp.maximum(x1r, x1c), 0.0)
        h = jnp.maximum(jnp.minimum(y2r, y2c) - jnp.maximum(y1r, y1c), 0.0)
        inter = w * h
        union = area_r + area_c - inter
        iou = inter / (union + 1e-06)

        col_ids = c0 + jax.lax.broadcasted_iota(
            jnp.int32, (ROW_TILE, COL_CHUNK), 1)
        iou = jnp.where(jnp.logical_or(row_ids == col_ids,
                                       col_ids >= N_KEEP), 0.0, iou)

        D_plane = D_plane + iou

        @pl.when(do_mlp_ref[i, chunk] != 0)
        def _():
            c_c = classesT_ref[0:1, c0:c0 + COL_CHUNK] * jnp.ones(
                (ROW_TILE, COL_CHUNK), jnp.int32)
            dx1 = jnp.abs(x1r - x1c)
            dy1 = jnp.abs(y1r - y1c)
            dx2 = jnp.abs(x2r - x2c)
            dy2 = jnp.abs(y2r - y2c)

            h2 = [wt(OFF_B2 + m) for m in range(16)]
            for k in range(32):
                t = wt(OFF_B1 + k)
                t = t + wt(OFF_W1 + 0 * 32 + k) * iou
                t = t + wt(OFF_W1 + 1 * 32 + k) * dx1
                t = t + wt(OFF_W1 + 2 * 32 + k) * dy1
                t = t + wt(OFF_W1 + 3 * 32 + k) * dx2
                t = t + wt(OFF_W1 + 4 * 32 + k) * dy2
                t = t + wt(OFF_W1 + 5 * 32 + k) * s_r
                t = t + wt(OFF_W1 + 6 * 32 + k) * s_c
                h1k = jnp.maximum(t, 0.0)
                for m in range(16):
                    h2[m] = h2[m] + wt(OFF_W2 + k * 16 + m) * h1k
            s_pre = wt(OFF_B3)
            for m in range(16):
                s_pre = s_pre + wt(OFF_W3 + m) * jnp.maximum(h2[m], 0.0)
            s_ij = jax.nn.sigmoid(s_pre)

            mask = jnp.logical_and(c_r == c_c, s_c > s_r)
            contrib = jnp.where(mask, s_ij * iou, 0.0)
            s_plane_ref[...] += contrib

    D = jnp.sum(D_plane, axis=1, keepdims=True) * (1.0 / N_KEEP)
    S = jnp.sum(s_plane_ref[...], axis=1, keepdims=True)

    lam_cols = (x1r, y1r, x2r, y2r, s_r)
    lam_pre = wt(OFF_LB2)
    for t in range(16):
        a = wt(OFF_LB1 + t)
        for c in range(5):
            a = a + wt(OFF_L1 + c * 16 + t) * lam_cols[c]
        lam_pre = lam_pre + wt(OFF_L2 + t) * jnp.maximum(a, 0.0)
    lam = jax.nn.sigmoid(lam_pre[:, 0:1])

    E = lam * S * D
    new_s = scores_r_ref[...] * jnp.exp(-E)
    out_ref[...] = jnp.where(row_ids[:, 0:1] < N_KEEP, new_s, -1.0)


NW = 16
EPW = N_PAD // NW
PAD_ROW0 = N_KEEP - (NW - 1) * EPW


def _sc_group_kernel(idx_hbm, scores_hbm, packed_hbm,
                     out_hbm,
                     idx_v, sc_v, cls_v, pk2, hist, lanepref, runhist,
                     mycnt, cc_v, cbase, dest_v, cc_sh, sem, sem2):
    wid = lax.axis_index("s")
    base = wid * EPW

    pltpu.sync_copy(idx_hbm.at[pl.ds(base, EPW)], idx_v)
    pltpu.sync_copy(scores_hbm.at[pl.ds(base, EPW)], sc_v)
    pltpu.async_copy(packed_hbm.at[idx_v], pk2, sem).wait()

    for v in range(2048 // 16):
        hist[pl.ds(v * 16, 16)] = jnp.zeros((16,), jnp.int32)
        runhist[pl.ds(v * 16, 16)] = jnp.zeros((16,), jnp.int32)

    lane = lax.iota(jnp.int32, 16)
    ones16 = jnp.ones((16,), jnp.int32)
    four16 = jnp.full((16,), 4, jnp.int32)
    for v in range(EPW // 16):
        sl = pl.ds(v * 16, 16)
        e16 = v * 16 + lane
        pos = base + e16
        mpad = pos >= N_KEEP
        clsf = plsc.load_gather(pk2, [e16, four16])
        c16 = jnp.where(mpad, 127, clsf.astype(jnp.int32))
        cls_v[sl] = c16
        sc_v[sl] = jnp.where(mpad, -1.0, sc_v[sl])
        plsc.addupdate_scatter(hist, [lane * 128 + c16], ones16)

    for cb in range(128 // 16):
        acc = jnp.zeros((16,), jnp.int32)
        for l in range(16):
            lanepref[pl.ds(l * 128 + cb * 16, 16)] = acc
            acc = acc + hist[pl.ds(l * 128 + cb * 16, 16)]
        mycnt[pl.ds(cb * 16, 16)] = acc

    pltpu.sync_copy(mycnt, cc_sh.at[wid])
    plsc.subcore_barrier()
    pltpu.sync_copy(cc_sh, cc_v)

    carry = jnp.int32(0)
    for cb in range(128 // 16):
        tot_cb = jnp.zeros((16,), jnp.int32)
        pw_cb = jnp.zeros((16,), jnp.int32)
        for w in range(NW):
            row = cc_v[w, pl.ds(cb * 16, 16)]
            tot_cb = tot_cb + row
            pw_cb = pw_cb + row * jnp.int32(w < wid)
        incl = plsc.cumsum(tot_cb)
        excl = incl - tot_cb + carry
        cbase[pl.ds(cb * 16, 16)] = excl + pw_cb
        carry = carry + jnp.sum(tot_cb)

    for v in range(EPW // 16):
        c16 = cls_v[pl.ds(v * 16, 16)]
        lidx = lane * 128 + c16
        base16 = plsc.load_gather(cbase, [c16])
        lp16 = plsc.load_gather(lanepref, [lidx])
        rl16 = plsc.load_gather(runhist, [lidx])
        dest_v[pl.ds(v * 16, 16)] = base16 + lp16 + rl16
        plsc.addupdate_scatter(runhist, [lidx], ones16)

    five16 = jnp.full((16,), 5, jnp.int32)
    for v in range(EPW // 16):
        sl = pl.ds(v * 16, 16)
        e16 = v * 16 + lane
        plsc.store_scatter(pk2, [e16, four16],
                           cls_v[sl].astype(jnp.float32))
        plsc.store_scatter(pk2, [e16, five16], sc_v[sl])

    pltpu.async_copy(pk2, out_hbm.at[dest_v], sem2).wait()


def _sc_group(idx, scores_k, classes, boxes):
    idx_p = jnp.pad(idx, (0, N_PAD - N_KEEP))
    scores_p = jnp.pad(scores_k, (0, N_PAD - N_KEEP))
    packed = jnp.concatenate(
        [boxes, classes.astype(jnp.float32)[:, None],
         jnp.zeros((boxes.shape[0], 3), jnp.float32)], axis=1)
    out2 = pl.kernel(
        _sc_group_kernel,
        out_type=jax.ShapeDtypeStruct((N_PAD, 8), jnp.float32),
        mesh=plsc.VectorSubcoreMesh(core_axis_name="c",
                                    subcore_axis_name="s",
                                    num_cores=1, num_subcores=NW),
        compiler_params=pltpu.CompilerParams(needs_layout_passes=False,
                                             use_tc_tiling_on_sc=False),
        scratch_types=[
            pltpu.VMEM((EPW,), jnp.int32),
            pltpu.VMEM((EPW,), jnp.float32),
            pltpu.VMEM((EPW,), jnp.int32),
            pltpu.VMEM((EPW, 8), jnp.float32),
            pltpu.VMEM((2048,), jnp.int32),
            pltpu.VMEM((2048,), jnp.int32),
            pltpu.VMEM((2048,), jnp.int32),
            pltpu.VMEM((128,), jnp.int32),
            pltpu.VMEM((NW, 128), jnp.int32),
            pltpu.VMEM((128,), jnp.int32),
            pltpu.VMEM((EPW,), jnp.int32),
            pltpu.VMEM_SHARED((NW, 128), jnp.int32),
            pltpu.SemaphoreType.DMA,
            pltpu.SemaphoreType.DMA,
        ],
    )(idx_p, scores_p, packed)
    boxes_s = out2[:, :4]
    classes_s = out2[:, 4].astype(jnp.int32)
    scores_s = out2[:, 5]
    return boxes_s, scores_s, classes_s


@jax.jit
def _dense_stage(boxes_p, scores_p, classes_p,
                 W1, b1, W2, b2, W3, b3, L1, lb1, L2, lb2):
    cls_first = classes_p[::ROW_TILE]
    cls_last = classes_p[ROW_TILE - 1::ROW_TILE]
    ws = jnp.searchsorted(classes_p, cls_first, side="left")
    we = jnp.searchsorted(classes_p, cls_last, side="right")
    c_lo = jnp.arange(N_CC) * COL_CHUNK
    c_hi = c_lo + COL_CHUNK
    do_mlp = jnp.logical_and(c_hi[None, :] > ws[:, None],
                             c_lo[None, :] < we[:, None]).astype(jnp.int32)

    vals = jnp.concatenate([
        W1.reshape(-1), b1, W2.reshape(-1), b2, W3.reshape(-1), b3,
        L1.reshape(-1), lb1, L2.reshape(-1), lb2])
    tbl = jnp.broadcast_to(vals[:, None, None],
                           (N_TBL, ROW_TILE, COL_CHUNK))

    boxesT = boxes_p.T
    scores_r = scores_p[:, None]
    scoresT = scores_p[None, :]
    classes_r = classes_p[:, None]
    classesT = classes_p[None, :]

    grid = (N_RT,)
    row_spec2 = lambda w: pl.BlockSpec((ROW_TILE, w), lambda i: (i, 0))
    full = lambda a, b: pl.BlockSpec((a, b), lambda i: (0, 0))
    smem = pl.BlockSpec(memory_space=pltpu.SMEM)

    out = pl.pallas_call(
        _dense_kernel,
        grid=grid,
        in_specs=[
            smem,
            row_spec2(4),
            full(4, N_PAD),
            row_spec2(1),
            full(1, N_PAD),
            row_spec2(1),
            full(1, N_PAD),
            pl.BlockSpec((N_TBL, ROW_TILE, COL_CHUNK),
                         lambda i: (0, 0, 0)),
        ],
        out_specs=pl.BlockSpec((ROW_TILE, 1), lambda i: (i, 0)),
        out_shape=jax.ShapeDtypeStruct((N_PAD, 1), jnp.float32),
        scratch_shapes=[pltpu.VMEM((ROW_TILE, COL_CHUNK), jnp.float32)],
    )(do_mlp, boxes_p, boxesT, scores_r, scoresT, classes_r, classesT, tbl)
    return out[:N_KEEP, 0]


def kernel(boxes, scores, classes, W1, b1, W2, b2, W3, b3, L1, lb1, L2, lb2):
    scores_k, idx = jax.lax.top_k(scores, N_KEEP)

    boxes_s, scores_s, classes_s = _sc_group(idx, scores_k, classes, boxes)

    new_scores = _dense_stage(boxes_s, scores_s, classes_s,
                              W1, b1, W2, b2, W3, b3, L1, lb1, L2, lb2)
    _, idx2 = jax.lax.top_k(new_scores, 50)
    return (boxes_s[idx2], new_scores[idx2], classes_s[idx2])
